# Initial kernel scaffold; baseline (speedup 1.0000x reference)
import jax
import jax.numpy as jnp
from jax.experimental import pallas as pl


def kernel(signals, feat, keys, level_table, W_feat, b_feat, W_mfcc, b_mfcc):
    raise NotImplementedError("write your pallas kernel here")


# speedup vs baseline: 1.8164x; 1.8164x over previous
import functools

import jax
import jax.numpy as jnp
import numpy as np
from jax.experimental import pallas as pl

NUM_CHANNEL = 4
NGRAM_SIZE = 3
LEVELS = 100
DIM = 10000
SEQ_LEN = 256
CHOSEN_FEAT = [547, 548, 549, 551, 554, 556, 557, 558, 559, 560, 561, 562,
               563, 565, 566, 567, 570, 576, 580, 581, 582, 583, 584, 585,
               588, 593, 598, 599, 600]

BLK = 1024
WEXT = BLK + 128
NBLK = pl.cdiv(DIM, BLK)
PADW = (NBLK - 1) * BLK + WEXT


def _encoder_kernel(sig_ref, ltpad_ref, keyspad_ref, vals_ref, wf_ref,
                    bf_ref, mf_ref, wm_ref, bm_ref, out_ref):
    i = pl.program_id(0)
    d0 = i * BLK

    lt_win = ltpad_ref[:, pl.ds(d0, WEXT)].astype(jnp.bfloat16)
    keys_win = keyspad_ref[:, pl.ds(d0, WEXT)]
    sig = sig_ref[...]
    idx = jnp.clip(jnp.floor(sig * LEVELS).astype(jnp.int32), 0, LEVELS - 1)
    iota_l = jax.lax.broadcasted_iota(jnp.int32, (SEQ_LEN, LEVELS), 1)
    ch = jnp.zeros((SEQ_LEN, WEXT), jnp.float32)
    for c in range(NUM_CHANNEL):
        oh = (idx[:, c][:, None] == iota_l).astype(jnp.bfloat16)
        part = jax.lax.dot_general(
            oh, lt_win, (((1,), (0,)), ((), ())),
            preferred_element_type=jnp.float32)
        ch = ch + part * keys_win[c][None, :]

    tt = SEQ_LEN - NGRAM_SIZE + 1
    a = ch[0:tt, 0:BLK]
    b = ch[1:tt + 1, 1:BLK + 1]
    cc = ch[2:tt + 2, 2:BLK + 2]
    sample_hv = jnp.sum(a * b * cc, axis=0)

    proj = vals_ref[...] * wf_ref[...]
    fhv = jnp.cos(proj + bf_ref[...]) * jnp.sin(proj)

    wm = wm_ref[...].astype(jnp.bfloat16)
    mf = mf_ref[...].astype(jnp.bfloat16)
    mrows = []
    for e in range(6):
        r = jax.lax.dot_general(wm[e], mf[e][:, None],
                                (((1,), (0,)), ((), ())),
                                preferred_element_type=jnp.float32)
        mrows.append(r[:, 0])
    mproj = jnp.stack(mrows)
    mhv = jnp.cos(mproj + bm_ref[...]) * jnp.sin(mproj)
    mfcc_hv = mhv[0] * mhv[1] * mhv[2] * mhv[3] * mhv[4] * mhv[5]

    f = {cf: fhv[j] for j, cf in enumerate(CHOSEN_FEAT)}
    expr = (f[547] * f[559] * f[565]
            + f[548] * f[560] * f[566]
            + f[549] * f[561] * f[567]
            + f[551] * f[554]
            + f[556] * f[558] * f[584] * f[557] * f[585] * f[581] * f[580]
            * f[582] * f[583] * f[598] * f[600] * f[599]
            + f[562] + f[563]
            + f[570] * f[588]
            + f[576] + f[593]
            + mfcc_hv)

    out = sample_hv * expr
    out_ref[0, :] = jnp.where(out > 0, 1.0, -1.0)


@functools.partial(jax.jit, static_argnames=("interpret",))
def _run(signals, feat, keys, level_table, W_feat, b_feat, W_mfcc, b_mfcc,
         interpret=False):
    def pad(tab):
        zeros = jnp.zeros(tab.shape[:-1] + (PADW - DIM - 2,), tab.dtype)
        return jnp.concatenate([tab[..., -2:], tab, zeros], axis=-1)

    ltpad = pad(level_table)
    keyspad = pad(keys)
    sel = np.array([cf - 1 for cf in CHOSEN_FEAT])
    vals = feat[sel][:, None]
    mf = feat[: 6 * 91].reshape(6, 91)

    out = pl.pallas_call(
        _encoder_kernel,
        grid=(NBLK,),
        in_specs=[
            pl.BlockSpec((SEQ_LEN, NUM_CHANNEL), lambda i: (0, 0)),
            pl.BlockSpec((LEVELS, PADW), lambda i: (0, 0)),
            pl.BlockSpec((NUM_CHANNEL, PADW), lambda i: (0, 0)),
            pl.BlockSpec((len(CHOSEN_FEAT), 1), lambda i: (0, 0)),
            pl.BlockSpec((len(CHOSEN_FEAT), BLK), lambda i: (0, i)),
            pl.BlockSpec((len(CHOSEN_FEAT), BLK), lambda i: (0, i)),
            pl.BlockSpec((6, 91), lambda i: (0, 0)),
            pl.BlockSpec((6, BLK, 91), lambda i: (0, i, 0)),
            pl.BlockSpec((6, BLK), lambda i: (0, i)),
        ],
        out_specs=pl.BlockSpec((1, BLK), lambda i: (0, i)),
        out_shape=jax.ShapeDtypeStruct((1, DIM), jnp.float32),
        interpret=interpret,
    )(signals, ltpad, keyspad, vals, W_feat, b_feat, mf, W_mfcc, b_mfcc)
    return out.reshape(-1)


def kernel(signals, feat, keys, level_table, W_feat, b_feat, W_mfcc, b_mfcc):
    return _run(signals, feat, keys, level_table, W_feat, b_feat,
                W_mfcc, b_mfcc)


# speedup vs baseline: 1.8661x; 1.0274x over previous
import functools

import jax
import jax.numpy as jnp
import numpy as np
from jax.experimental import pallas as pl

NUM_CHANNEL = 4
NGRAM_SIZE = 3
LEVELS = 100
DIM = 10000
SEQ_LEN = 256
CHOSEN_FEAT = [547, 548, 549, 551, 554, 556, 557, 558, 559, 560, 561, 562,
               563, 565, 566, 567, 570, 576, 580, 581, 582, 583, 584, 585,
               588, 593, 598, 599, 600]

BLK = 2048
WEXT = BLK + 128
NBLK = pl.cdiv(DIM, BLK)
PADW = (NBLK - 1) * BLK + WEXT


def _encoder_kernel(sig_ref, ltpad_ref, keyspad_ref, vals_ref, wf_ref,
                    bf_ref, mf_ref, wm_ref, bm_ref, out_ref):
    i = pl.program_id(0)
    d0 = i * BLK

    lt_win = ltpad_ref[:, pl.ds(d0, WEXT)].astype(jnp.bfloat16)
    keys_win = keyspad_ref[:, pl.ds(d0, WEXT)]
    sig = sig_ref[...]
    idx = jnp.clip(jnp.floor(sig * LEVELS).astype(jnp.int32), 0, LEVELS - 1)
    iota_l = jax.lax.broadcasted_iota(jnp.int32, (SEQ_LEN, LEVELS), 1)
    ch = jnp.zeros((SEQ_LEN, WEXT), jnp.float32)
    for c in range(NUM_CHANNEL):
        oh = (idx[:, c][:, None] == iota_l).astype(jnp.bfloat16)
        part = jax.lax.dot_general(
            oh, lt_win, (((1,), (0,)), ((), ())),
            preferred_element_type=jnp.float32)
        ch = ch + part * keys_win[c][None, :]

    tt = SEQ_LEN - NGRAM_SIZE + 1
    a = ch[0:tt, 0:BLK]
    b = ch[1:tt + 1, 1:BLK + 1]
    cc = ch[2:tt + 2, 2:BLK + 2]
    sample_hv = jnp.sum(a * b * cc, axis=0)

    proj = vals_ref[...] * wf_ref[...]
    fhv = jnp.cos(proj + bf_ref[...]) * jnp.sin(proj)

    wm = wm_ref[...].astype(jnp.bfloat16)
    mf = mf_ref[...].astype(jnp.bfloat16)
    mrows = []
    for e in range(6):
        r = jax.lax.dot_general(wm[e], mf[e][:, None],
                                (((1,), (0,)), ((), ())),
                                preferred_element_type=jnp.float32)
        mrows.append(r[:, 0])
    mproj = jnp.stack(mrows)
    mhv = jnp.cos(mproj + bm_ref[...]) * jnp.sin(mproj)
    mfcc_hv = mhv[0] * mhv[1] * mhv[2] * mhv[3] * mhv[4] * mhv[5]

    f = {cf: fhv[j] for j, cf in enumerate(CHOSEN_FEAT)}
    expr = (f[547] * f[559] * f[565]
            + f[548] * f[560] * f[566]
            + f[549] * f[561] * f[567]
            + f[551] * f[554]
            + f[556] * f[558] * f[584] * f[557] * f[585] * f[581] * f[580]
            * f[582] * f[583] * f[598] * f[600] * f[599]
            + f[562] + f[563]
            + f[570] * f[588]
            + f[576] + f[593]
            + mfcc_hv)

    out = sample_hv * expr
    out_ref[0, :] = jnp.where(out > 0, 1.0, -1.0)


@functools.partial(jax.jit, static_argnames=("interpret",))
def _run(signals, feat, keys, level_table, W_feat, b_feat, W_mfcc, b_mfcc,
         interpret=False):
    def pad(tab):
        zeros = jnp.zeros(tab.shape[:-1] + (PADW - DIM - 2,), tab.dtype)
        return jnp.concatenate([tab[..., -2:], tab, zeros], axis=-1)

    ltpad = pad(level_table)
    keyspad = pad(keys)
    sel = np.array([cf - 1 for cf in CHOSEN_FEAT])
    vals = feat[sel][:, None]
    mf = feat[: 6 * 91].reshape(6, 91)

    out = pl.pallas_call(
        _encoder_kernel,
        grid=(NBLK,),
        in_specs=[
            pl.BlockSpec((SEQ_LEN, NUM_CHANNEL), lambda i: (0, 0)),
            pl.BlockSpec((LEVELS, PADW), lambda i: (0, 0)),
            pl.BlockSpec((NUM_CHANNEL, PADW), lambda i: (0, 0)),
            pl.BlockSpec((len(CHOSEN_FEAT), 1), lambda i: (0, 0)),
            pl.BlockSpec((len(CHOSEN_FEAT), BLK), lambda i: (0, i)),
            pl.BlockSpec((len(CHOSEN_FEAT), BLK), lambda i: (0, i)),
            pl.BlockSpec((6, 91), lambda i: (0, 0)),
            pl.BlockSpec((6, BLK, 91), lambda i: (0, i, 0)),
            pl.BlockSpec((6, BLK), lambda i: (0, i)),
        ],
        out_specs=pl.BlockSpec((1, BLK), lambda i: (0, i)),
        out_shape=jax.ShapeDtypeStruct((1, DIM), jnp.float32),
        interpret=interpret,
    )(signals, ltpad, keyspad, vals, W_feat, b_feat, mf, W_mfcc, b_mfcc)
    return out.reshape(-1)


def kernel(signals, feat, keys, level_table, W_feat, b_feat, W_mfcc, b_mfcc):
    return _run(signals, feat, keys, level_table, W_feat, b_feat,
                W_mfcc, b_mfcc)


# speedup vs baseline: 2.0012x; 1.0724x over previous
import functools

import jax
import jax.numpy as jnp
import numpy as np
from jax.experimental import pallas as pl

NUM_CHANNEL = 4
NGRAM_SIZE = 3
LEVELS = 100
DIM = 10000
SEQ_LEN = 256
CHOSEN_FEAT = [547, 548, 549, 551, 554, 556, 557, 558, 559, 560, 561, 562,
               563, 565, 566, 567, 570, 576, 580, 581, 582, 583, 584, 585,
               588, 593, 598, 599, 600]

BLK = 2048
WEXT = BLK + 128
NBLK = pl.cdiv(DIM, BLK)
PADW = (NBLK - 1) * BLK + WEXT


def _encoder_kernel(sig_ref, ltpad_ref, keyspad_ref, vals_ref, wf_ref,
                    bf_ref, mf_ref, wm_ref, bm_ref, out_ref):
    i = pl.program_id(0)
    d0 = i * BLK

    lt_win = ltpad_ref[:, pl.ds(d0, WEXT)]
    keys_win = keyspad_ref[:, pl.ds(d0, WEXT)]
    sig = sig_ref[...]
    idx = jnp.clip(jnp.floor(sig * LEVELS).astype(jnp.int32), 0, LEVELS - 1)
    iota_l = jax.lax.broadcasted_iota(jnp.int32, (SEQ_LEN, LEVELS), 1)
    parts = []
    for c in range(NUM_CHANNEL):
        oh = (idx[:, c][:, None] == iota_l).astype(jnp.bfloat16)
        eff = lt_win * keys_win[c][None, :]
        parts.append(jax.lax.dot_general(
            oh, eff, (((1,), (0,)), ((), ())),
            preferred_element_type=jnp.float32))
    ch = ((parts[0] + parts[1]) + (parts[2] + parts[3])
          ).astype(jnp.bfloat16)

    tt = SEQ_LEN - NGRAM_SIZE + 1
    a = ch[0:tt, 0:BLK]
    b = ch[1:tt + 1, 1:BLK + 1]
    cc = ch[2:tt + 2, 2:BLK + 2]
    grams = a * b * cc
    ones = jnp.ones((1, tt), jnp.bfloat16)
    sample_hv = jax.lax.dot_general(
        ones, grams, (((1,), (0,)), ((), ())),
        preferred_element_type=jnp.float32)[0]

    proj = vals_ref[...] * wf_ref[...]
    fhv = jnp.cos(proj + bf_ref[...]) * jnp.sin(proj)

    wm = wm_ref[...].astype(jnp.bfloat16)
    mf = mf_ref[...].astype(jnp.bfloat16)
    mrows = []
    for e in range(6):
        r = jax.lax.dot_general(wm[e], mf[e][:, None],
                                (((1,), (0,)), ((), ())),
                                preferred_element_type=jnp.float32)
        mrows.append(r[:, 0])
    mproj = jnp.stack(mrows)
    mhv = jnp.cos(mproj + bm_ref[...]) * jnp.sin(mproj)
    mfcc_hv = mhv[0] * mhv[1] * mhv[2] * mhv[3] * mhv[4] * mhv[5]

    f = {cf: fhv[j] for j, cf in enumerate(CHOSEN_FEAT)}
    expr = (f[547] * f[559] * f[565]
            + f[548] * f[560] * f[566]
            + f[549] * f[561] * f[567]
            + f[551] * f[554]
            + f[556] * f[558] * f[584] * f[557] * f[585] * f[581] * f[580]
            * f[582] * f[583] * f[598] * f[600] * f[599]
            + f[562] + f[563]
            + f[570] * f[588]
            + f[576] + f[593]
            + mfcc_hv)

    out = sample_hv * expr
    out_ref[0, :] = jnp.where(out > 0, 1.0, -1.0)


@functools.partial(jax.jit, static_argnames=("interpret",))
def _run(signals, feat, keys, level_table, W_feat, b_feat, W_mfcc, b_mfcc,
         interpret=False):
    def pad(tab):
        tab = tab.astype(jnp.bfloat16)
        zeros = jnp.zeros(tab.shape[:-1] + (PADW - DIM - 2,), tab.dtype)
        return jnp.concatenate([tab[..., -2:], tab, zeros], axis=-1)

    ltpad = pad(level_table)
    keyspad = pad(keys)
    sel = np.array([cf - 1 for cf in CHOSEN_FEAT])
    vals = feat[sel][:, None]
    mf = feat[: 6 * 91].reshape(6, 91)

    out = pl.pallas_call(
        _encoder_kernel,
        grid=(NBLK,),
        in_specs=[
            pl.BlockSpec((SEQ_LEN, NUM_CHANNEL), lambda i: (0, 0)),
            pl.BlockSpec((LEVELS, PADW), lambda i: (0, 0)),
            pl.BlockSpec((NUM_CHANNEL, PADW), lambda i: (0, 0)),
            pl.BlockSpec((len(CHOSEN_FEAT), 1), lambda i: (0, 0)),
            pl.BlockSpec((len(CHOSEN_FEAT), BLK), lambda i: (0, i)),
            pl.BlockSpec((len(CHOSEN_FEAT), BLK), lambda i: (0, i)),
            pl.BlockSpec((6, 91), lambda i: (0, 0)),
            pl.BlockSpec((6, BLK, 91), lambda i: (0, i, 0)),
            pl.BlockSpec((6, BLK), lambda i: (0, i)),
        ],
        out_specs=pl.BlockSpec((1, BLK), lambda i: (0, i)),
        out_shape=jax.ShapeDtypeStruct((1, DIM), jnp.float32),
        interpret=interpret,
    )(signals, ltpad, keyspad, vals, W_feat, b_feat, mf, W_mfcc, b_mfcc)
    return out.reshape(-1)


def kernel(signals, feat, keys, level_table, W_feat, b_feat, W_mfcc, b_mfcc):
    return _run(signals, feat, keys, level_table, W_feat, b_feat,
                W_mfcc, b_mfcc)
